# Initial kernel scaffold; baseline (speedup 1.0000x reference)
import jax
import jax.numpy as jnp
from jax.experimental import pallas as pl


def kernel(original_vertices, original_faces, simplified_vertices, simplified_faces, sample_prob):
    raise NotImplementedError("write your pallas kernel here")


# speedup vs baseline: 1.0233x; 1.0233x over previous
import jax
import jax.numpy as jnp
from jax.experimental import pallas as pl
from jax.experimental.pallas import tpu as pltpu

_S = 4096
_BLK = 512
_NP = 10000


def _sample_points(key, vertices, faces):
    tri = vertices[faces]
    v0f, v1f, v2f = tri[:, 0], tri[:, 1], tri[:, 2]
    areas = jnp.linalg.norm(jnp.cross(v1f - v0f, v2f - v0f), axis=1) * 0.5
    probs = areas / areas.sum()
    k1, k2, k3 = jax.random.split(key, 3)
    p_cuml = jnp.cumsum(probs)
    r = p_cuml[-1] * (1.0 - jax.random.uniform(k1, (_S,), dtype=jnp.float32))
    sel = jnp.searchsorted(p_cuml, r).astype(jnp.int32)
    u = jax.random.uniform(k2, (_S,), dtype=jnp.float32)
    v = jax.random.uniform(k3, (_S,), dtype=jnp.float32)
    w = 1.0 - u - v
    mask = w < 0
    u = jnp.where(mask, u + w, u)
    v = jnp.where(mask, 1.0 - u, v)
    w = jnp.where(mask, 0.0, w)
    st = vertices[faces[sel]]
    return u[:, None] * st[:, 0] + v[:, None] * st[:, 1] + w[:, None] * st[:, 2]


def _dist_body(ot_ref, st_ref, prob_ref, out_ref, colmin_ref, osum_ref):
    i = pl.program_id(0)
    ob = ot_ref[...]
    st = st_ref[...]
    on = jnp.sum(ob * ob, axis=0)
    sn = jnp.sum(st * st, axis=0)
    cross = jax.lax.dot_general(
        ob, st, (((0,), (0,)), ((), ())),
        preferred_element_type=jnp.float32,
        precision=jax.lax.Precision.HIGHEST)
    d2 = jnp.maximum(on[:, None] + sn[None, :] - 2.0 * cross, 0.0)

    @pl.when(i == 0)
    def _init():
        colmin_ref[...] = jnp.full((_S,), jnp.inf, jnp.float32)
        osum_ref[0] = 0.0

    colmin_ref[...] = jnp.minimum(colmin_ref[...], jnp.min(d2, axis=0))
    osum_ref[0] += jnp.sum(jnp.sqrt(jnp.min(d2, axis=1)))

    @pl.when(i == pl.num_programs(0) - 1)
    def _fin():
        mean_o2s = osum_ref[0] / _S
        mean_s2o = jnp.sum(jnp.sqrt(colmin_ref[...])) / _S
        mean_p = jnp.sum(prob_ref[...]) / _NP
        out_ref[0, 0] = mean_s2o * mean_p + mean_o2s


def _min_dist_loss(ot, st, prob_pad, interpret=False):
    return pl.pallas_call(
        _dist_body,
        grid=(_S // _BLK,),
        in_specs=[
            pl.BlockSpec((3, _BLK), lambda i: (0, i)),
            pl.BlockSpec((3, _S), lambda i: (0, 0)),
            pl.BlockSpec((80, 128), lambda i: (0, 0)),
        ],
        out_specs=pl.BlockSpec(memory_space=pltpu.SMEM),
        out_shape=jax.ShapeDtypeStruct((1, 1), jnp.float32),
        scratch_shapes=[
            pltpu.VMEM((_S,), jnp.float32),
            pltpu.SMEM((1,), jnp.float32),
        ],
        interpret=interpret,
    )(ot, st, prob_pad)


def kernel(original_vertices, original_faces, simplified_vertices,
           simplified_faces, sample_prob):
    os_pts = _sample_points(jax.random.key(1), original_vertices, original_faces)
    ss_pts = _sample_points(jax.random.key(2), simplified_vertices, simplified_faces)
    prob_pad = jnp.concatenate(
        [sample_prob, jnp.zeros((80 * 128 - _NP,), jnp.float32)]).reshape(80, 128)
    loss = _min_dist_loss(os_pts.T, ss_pts.T, prob_pad)
    return loss[0, 0]


# speedup vs baseline: 5.3663x; 5.2442x over previous
import functools

import jax
import jax.numpy as jnp
from jax import lax
from jax.experimental import pallas as pl
from jax.experimental.pallas import tpu as pltpu
from jax.experimental.pallas import tpu_sc as plsc

_S = 4096
_BLK = 512
_NP = 10000
_NW = 32

_FO, _PWO = 200000, 6272
_FS, _PWS = 20000, 640
_GROUP = 32


def _compute_group_areas(g, bufs, cum_ref):
    for sub in range(2):
        v = [[bufs[c][pl.ds(k * 32 + sub * 16, 16)] for c in range(3)]
             for k in range(3)]
        e1 = [v[1][c] - v[0][c] for c in range(3)]
        e2 = [v[2][c] - v[0][c] for c in range(3)]
        nx = e1[1] * e2[2] - e1[2] * e2[1]
        ny = e1[2] * e2[0] - e1[0] * e2[2]
        nz = e1[0] * e2[1] - e1[1] * e2[0]
        q = nx * nx + ny * ny + nz * nz
        y = plsc.bitcast(jnp.int32(0x5F3759DF) - (plsc.bitcast(q, jnp.int32) >> 1),
                         jnp.float32)
        for _ in range(3):
            y = y * (1.5 - 0.5 * q * y * y)
        area = jnp.where(q > 0.0, 0.5 * q * y, 0.0)
        cum_ref[pl.ds(g * _GROUP + sub * 16, 16)] = area


def _area_section(faces_hbm, vcomp_hbm, cum_hbm, idx_v, bufs, sems, cum_v,
                  wid, n_per_wid):
    n_idx = 3 * n_per_wid
    n_groups = n_per_wid // _GROUP
    pltpu.sync_copy(faces_hbm.at[pl.ds(wid * n_idx, n_idx)],
                    idx_v.at[pl.ds(0, n_idx)])

    def issue(g, b):
        for c in range(3):
            pltpu.async_copy(vcomp_hbm[c].at[idx_v.at[pl.ds(g * 96, 96)]],
                             bufs[b][c], sems[b])

    def wait(b):
        for c in range(3):
            pltpu.make_async_copy(vcomp_hbm[c].at[pl.ds(0, 96)], bufs[b][c],
                                  sems[b]).wait()

    for b in range(2):
        issue(b, b)

    def outer(q, _):
        for b in range(2):
            g = q * 2 + b
            wait(b)
            _compute_group_areas(g, bufs[b], cum_v)

            @pl.when(g + 2 < n_groups)
            def _():
                issue(g + 2, b)
        return 0

    lax.fori_loop(0, n_groups // 2, outer, 0)

    def cs_body(i, carry):
        vv = cum_v[pl.ds(i * 16, 16)]
        c2 = jnp.cumsum(vv) + carry
        cum_v[pl.ds(i * 16, 16)] = c2
        return jnp.max(c2)

    lax.fori_loop(0, n_per_wid // 16, cs_body, jnp.float32(0.0))
    pltpu.sync_copy(cum_v.at[pl.ds(0, n_per_wid)],
                    cum_hbm.at[pl.ds(wid * n_per_wid, n_per_wid)])


def _sc_areas_body(fo_hbm, vxo, vyo, vzo, fs_hbm, vxs, vys, vzs,
                   cumo_hbm, cums_hbm,
                   idx_v, b0x, b0y, b0z, b1x, b1y, b1z, cum_v, sem0, sem1):
    wid = lax.axis_index("s") * 2 + lax.axis_index("c")
    bufs = ((b0x, b0y, b0z), (b1x, b1y, b1z))
    sems = (sem0, sem1)
    _area_section(fo_hbm, (vxo, vyo, vzo), cumo_hbm, idx_v, bufs, sems, cum_v,
                  wid, _PWO)
    _area_section(fs_hbm, (vxs, vys, vzs), cums_hbm, idx_v, bufs, sems, cum_v,
                  wid, _PWS)


def _sc_areas(fo_perm, vo_comp, fs_perm, vs_comp):
    mesh = plsc.VectorSubcoreMesh(core_axis_name="c", subcore_axis_name="s")
    return pl.kernel(
        _sc_areas_body,
        out_type=(jax.ShapeDtypeStruct((_NW * _PWO,), jnp.float32),
                  jax.ShapeDtypeStruct((_NW * _PWS,), jnp.float32)),
        mesh=mesh,
        compiler_params=pltpu.CompilerParams(needs_layout_passes=False),
        scratch_types=[
            pltpu.VMEM((3 * _PWO,), jnp.int32),
            pltpu.VMEM((96,), jnp.float32),
            pltpu.VMEM((96,), jnp.float32),
            pltpu.VMEM((96,), jnp.float32),
            pltpu.VMEM((96,), jnp.float32),
            pltpu.VMEM((96,), jnp.float32),
            pltpu.VMEM((96,), jnp.float32),
            pltpu.VMEM((_PWO,), jnp.float32),
            pltpu.SemaphoreType.DMA,
            pltpu.SemaphoreType.DMA,
        ],
    )(fo_perm, *vo_comp, fs_perm, *vs_comp)


def _select_faces(cum, n_per_wid, r_unit):
    co = cum.reshape(_NW, n_per_wid)
    tot = co[:, -1]
    off = jnp.concatenate([jnp.zeros((1,), jnp.float32), jnp.cumsum(tot)[:-1]])
    glob = (co + off[:, None]).reshape(-1)
    total = off[-1] + tot[-1]
    r = total * r_unit
    return jnp.searchsorted(glob, r).astype(jnp.int32)


def _sample_points(vertices, faces, sel, u, v):
    w = 1.0 - u - v
    mask = w < 0
    u = jnp.where(mask, u + w, u)
    v = jnp.where(mask, 1.0 - u, v)
    w = jnp.where(mask, 0.0, w)
    st = vertices[faces[sel]]
    return u[:, None] * st[:, 0] + v[:, None] * st[:, 1] + w[:, None] * st[:, 2]


def _dist_body(ot_ref, st_ref, prob_ref, out_ref, colmin_ref, osum_ref):
    i = pl.program_id(0)
    ob = ot_ref[...]
    st = st_ref[...]
    on = jnp.sum(ob * ob, axis=0)
    sn = jnp.sum(st * st, axis=0)
    cross = jax.lax.dot_general(
        ob, st, (((0,), (0,)), ((), ())),
        preferred_element_type=jnp.float32,
        precision=jax.lax.Precision.HIGHEST)
    d2 = jnp.maximum(on[:, None] + sn[None, :] - 2.0 * cross, 0.0)

    @pl.when(i == 0)
    def _init():
        colmin_ref[...] = jnp.full((_S,), jnp.inf, jnp.float32)
        osum_ref[0] = 0.0

    colmin_ref[...] = jnp.minimum(colmin_ref[...], jnp.min(d2, axis=0))
    osum_ref[0] += jnp.sum(jnp.sqrt(jnp.min(d2, axis=1)))

    @pl.when(i == pl.num_programs(0) - 1)
    def _fin():
        mean_o2s = osum_ref[0] / _S
        mean_s2o = jnp.sum(jnp.sqrt(colmin_ref[...])) / _S
        mean_p = jnp.sum(prob_ref[...]) / _NP
        out_ref[0, 0] = mean_s2o * mean_p + mean_o2s


def _min_dist_loss(ot, st, prob_pad, interpret=False):
    return pl.pallas_call(
        _dist_body,
        grid=(_S // _BLK,),
        in_specs=[
            pl.BlockSpec((3, _BLK), lambda i: (0, i)),
            pl.BlockSpec((3, _S), lambda i: (0, 0)),
            pl.BlockSpec((80, 128), lambda i: (0, 0)),
        ],
        out_specs=pl.BlockSpec(memory_space=pltpu.SMEM),
        out_shape=jax.ShapeDtypeStruct((1, 1), jnp.float32),
        scratch_shapes=[
            pltpu.VMEM((_S,), jnp.float32),
            pltpu.SMEM((1,), jnp.float32),
        ],
        interpret=interpret,
    )(ot, st, prob_pad)


def kernel(original_vertices, original_faces, simplified_vertices,
           simplified_faces, sample_prob):
    ko1, ko2, ko3 = jax.random.split(jax.random.key(1), 3)
    ks1, ks2, ks3 = jax.random.split(jax.random.key(2), 3)
    ro = 1.0 - jax.random.uniform(ko1, (_S,), dtype=jnp.float32)
    rs = 1.0 - jax.random.uniform(ks1, (_S,), dtype=jnp.float32)
    uo = jax.random.uniform(ko2, (_S,), dtype=jnp.float32)
    vo = jax.random.uniform(ko3, (_S,), dtype=jnp.float32)
    us = jax.random.uniform(ks2, (_S,), dtype=jnp.float32)
    vs = jax.random.uniform(ks3, (_S,), dtype=jnp.float32)

    vo_comp = tuple(original_vertices[:, c] for c in range(3))
    vs_comp = tuple(simplified_vertices[:, c] for c in range(3))
    fo_perm = jnp.pad(original_faces, ((0, _NW * _PWO - _FO), (0, 0))) \
        .reshape(-1, _GROUP, 3).transpose(0, 2, 1).reshape(-1)
    fs_perm = jnp.pad(simplified_faces, ((0, _NW * _PWS - _FS), (0, 0))) \
        .reshape(-1, _GROUP, 3).transpose(0, 2, 1).reshape(-1)

    cumo, cums = _sc_areas(fo_perm, vo_comp, fs_perm, vs_comp)

    sel_o = _select_faces(cumo, _PWO, ro)
    sel_s = _select_faces(cums, _PWS, rs)

    os_pts = _sample_points(original_vertices, original_faces, sel_o, uo, vo)
    ss_pts = _sample_points(simplified_vertices, simplified_faces, sel_s, us, vs)

    prob_pad = jnp.concatenate(
        [sample_prob, jnp.zeros((80 * 128 - _NP,), jnp.float32)]).reshape(80, 128)
    loss = _min_dist_loss(os_pts.T, ss_pts.T, prob_pad)
    return loss[0, 0]


# speedup vs baseline: 10.6761x; 1.9895x over previous
import functools

import jax
import jax.numpy as jnp
from jax import lax
from jax.experimental import pallas as pl
from jax.experimental.pallas import tpu as pltpu
from jax.experimental.pallas import tpu_sc as plsc

_S = 4096
_BLK = 512
_NP = 10000
_NW = 32

_FO, _PWO = 200000, 6272
_FS, _PWS = 20000, 640
_GROUP = 32


def _compute_group_areas(g, bufs, cum_ref):
    for sub in range(2):
        v = [[bufs[c][pl.ds(k * 32 + sub * 16, 16)] for c in range(3)]
             for k in range(3)]
        e1 = [v[1][c] - v[0][c] for c in range(3)]
        e2 = [v[2][c] - v[0][c] for c in range(3)]
        nx = e1[1] * e2[2] - e1[2] * e2[1]
        ny = e1[2] * e2[0] - e1[0] * e2[2]
        nz = e1[0] * e2[1] - e1[1] * e2[0]
        q = nx * nx + ny * ny + nz * nz
        y = plsc.bitcast(jnp.int32(0x5F3759DF) - (plsc.bitcast(q, jnp.int32) >> 1),
                         jnp.float32)
        for _ in range(3):
            y = y * (1.5 - 0.5 * q * y * y)
        area = jnp.where(q > 0.0, 0.5 * q * y, 0.0)
        cum_ref[pl.ds(g * _GROUP + sub * 16, 16)] = area


def _area_section(faces_hbm, vcomp_hbm, cum_hbm, idx_v, bufs, sems, cum_v,
                  wid, n_per_wid):
    n_idx = 3 * n_per_wid
    n_groups = n_per_wid // _GROUP
    pltpu.sync_copy(faces_hbm.at[pl.ds(wid * n_idx, n_idx)],
                    idx_v.at[pl.ds(0, n_idx)])

    def issue(g, b):
        for c in range(3):
            pltpu.async_copy(vcomp_hbm[c].at[idx_v.at[pl.ds(g * 96, 96)]],
                             bufs[b][c], sems[b])

    def wait(b):
        for c in range(3):
            pltpu.make_async_copy(vcomp_hbm[c].at[pl.ds(0, 96)], bufs[b][c],
                                  sems[b]).wait()

    for b in range(2):
        issue(b, b)

    def outer(q, _):
        for b in range(2):
            g = q * 2 + b
            wait(b)
            _compute_group_areas(g, bufs[b], cum_v)

            @pl.when(g + 2 < n_groups)
            def _():
                issue(g + 2, b)
        return 0

    lax.fori_loop(0, n_groups // 2, outer, 0)

    def cs_body(i, carry):
        vv = cum_v[pl.ds(i * 16, 16)]
        c2 = jnp.cumsum(vv) + carry
        cum_v[pl.ds(i * 16, 16)] = c2
        return jnp.max(c2)

    lax.fori_loop(0, n_per_wid // 16, cs_body, jnp.float32(0.0))
    pltpu.sync_copy(cum_v.at[pl.ds(0, n_per_wid)],
                    cum_hbm.at[pl.ds(wid * n_per_wid, n_per_wid)])


def _sc_areas_body(fo_hbm, vxo, vyo, vzo, fs_hbm, vxs, vys, vzs,
                   cumo_hbm, cums_hbm,
                   idx_v, b0x, b0y, b0z, b1x, b1y, b1z, cum_v, sem0, sem1):
    wid = lax.axis_index("s") * 2 + lax.axis_index("c")
    bufs = ((b0x, b0y, b0z), (b1x, b1y, b1z))
    sems = (sem0, sem1)
    _area_section(fo_hbm, (vxo, vyo, vzo), cumo_hbm, idx_v, bufs, sems, cum_v,
                  wid, _PWO)
    _area_section(fs_hbm, (vxs, vys, vzs), cums_hbm, idx_v, bufs, sems, cum_v,
                  wid, _PWS)


def _sc_areas(fo_perm, vo_comp, fs_perm, vs_comp):
    mesh = plsc.VectorSubcoreMesh(core_axis_name="c", subcore_axis_name="s")
    return pl.kernel(
        _sc_areas_body,
        out_type=(jax.ShapeDtypeStruct((_NW * _PWO,), jnp.float32),
                  jax.ShapeDtypeStruct((_NW * _PWS,), jnp.float32)),
        mesh=mesh,
        compiler_params=pltpu.CompilerParams(needs_layout_passes=False),
        scratch_types=[
            pltpu.VMEM((3 * _PWO,), jnp.int32),
            pltpu.VMEM((96,), jnp.float32),
            pltpu.VMEM((96,), jnp.float32),
            pltpu.VMEM((96,), jnp.float32),
            pltpu.VMEM((96,), jnp.float32),
            pltpu.VMEM((96,), jnp.float32),
            pltpu.VMEM((96,), jnp.float32),
            pltpu.VMEM((_PWO,), jnp.float32),
            pltpu.SemaphoreType.DMA,
            pltpu.SemaphoreType.DMA,
        ],
    )(fo_perm, *vo_comp, fs_perm, *vs_comp)


def _assemble_glob(cum, n_per_wid, r_unit):
    co = cum.reshape(_NW, n_per_wid)
    tot = co[:, -1]
    off = jnp.concatenate([jnp.zeros((1,), jnp.float32), jnp.cumsum(tot)[:-1]])
    glob = (co + off[:, None]).reshape(-1)
    total = off[-1] + tot[-1]
    return glob, total * r_unit


def _sample_section(glob_hbm, ftabs, vtabs, r_hbm, u_hbm, v_hbm, pts_hbm,
                    scr, sem, wid, nsteps, nfaces_pad):
    (r_v, u_v, v_v, lo_v, hi_v, mid_v, val_v, id0, id1, id2,
     cbufs, px_v, py_v, pz_v) = scr
    base = wid * 128
    pltpu.sync_copy(r_hbm.at[pl.ds(base, 128)], r_v)
    pltpu.sync_copy(u_hbm.at[pl.ds(base, 128)], u_v)
    pltpu.sync_copy(v_hbm.at[pl.ds(base, 128)], v_v)
    for j in range(8):
        s = pl.ds(j * 16, 16)
        lo_v[s] = jnp.zeros((16,), jnp.int32)
        hi_v[s] = jnp.full((16,), nfaces_pad, jnp.int32)

    def step(t, _):
        for j in range(8):
            s = pl.ds(j * 16, 16)
            mid_v[s] = (lo_v[s] + hi_v[s]) // 2
        pltpu.async_copy(glob_hbm.at[mid_v], val_v, sem).wait()
        for j in range(8):
            s = pl.ds(j * 16, 16)
            pred = val_v[s] < r_v[s]
            lo_v[s] = jnp.where(pred, mid_v[s] + 1, lo_v[s])
            hi_v[s] = jnp.where(pred, hi_v[s], mid_v[s])
        return 0

    lax.fori_loop(0, nsteps, step, 0)

    ids = (id0, id1, id2)
    for k in range(3):
        pltpu.async_copy(ftabs[k].at[lo_v], ids[k], sem)
    for k in range(3):
        pltpu.make_async_copy(ftabs[k].at[pl.ds(0, 128)], ids[k], sem).wait()
    for k in range(3):
        for c in range(3):
            pltpu.async_copy(vtabs[c].at[ids[k]], cbufs[k][c], sem)
    for k in range(3):
        for c in range(3):
            pltpu.make_async_copy(vtabs[c].at[pl.ds(0, 128)], cbufs[k][c],
                                  sem).wait()

    outs = (px_v, py_v, pz_v)
    for j in range(8):
        s = pl.ds(j * 16, 16)
        u = u_v[s]
        v = v_v[s]
        w = 1.0 - u - v
        mask = w < 0
        u = jnp.where(mask, u + w, u)
        v = jnp.where(mask, 1.0 - u, v)
        w = jnp.where(mask, 0.0, w)
        for c in range(3):
            outs[c][s] = (u * cbufs[0][c][s] + v * cbufs[1][c][s]
                          + w * cbufs[2][c][s])
    for c in range(3):
        pltpu.sync_copy(outs[c], pts_hbm.at[pl.ds(c * _S + base, 128)])


def _sc_sample_body(globo, f0o, f1o, f2o, vxo, vyo, vzo, ro, uo, vo,
                    globs, f0s, f1s, f2s, vxs, vys, vzs, rs, us, vs,
                    ptso_hbm, ptss_hbm,
                    r_v, u_v, v_v, lo_v, hi_v, mid_v, val_v, id0, id1, id2,
                    c00, c01, c02, c10, c11, c12, c20, c21, c22,
                    px_v, py_v, pz_v, sem):
    wid = lax.axis_index("s") * 2 + lax.axis_index("c")
    cbufs = ((c00, c01, c02), (c10, c11, c12), (c20, c21, c22))
    scr = (r_v, u_v, v_v, lo_v, hi_v, mid_v, val_v, id0, id1, id2,
           cbufs, px_v, py_v, pz_v)
    _sample_section(globo, (f0o, f1o, f2o), (vxo, vyo, vzo), ro, uo, vo,
                    ptso_hbm, scr, sem, wid, 18, _NW * _PWO)
    _sample_section(globs, (f0s, f1s, f2s), (vxs, vys, vzs), rs, us, vs,
                    ptss_hbm, scr, sem, wid, 15, _NW * _PWS)


def _sc_sample(args_o, args_s):
    mesh = plsc.VectorSubcoreMesh(core_axis_name="c", subcore_axis_name="s")
    f32 = jnp.float32
    i32 = jnp.int32
    return pl.kernel(
        _sc_sample_body,
        out_type=(jax.ShapeDtypeStruct((3 * _S,), f32),
                  jax.ShapeDtypeStruct((3 * _S,), f32)),
        mesh=mesh,
        compiler_params=pltpu.CompilerParams(needs_layout_passes=False),
        scratch_types=(
            [pltpu.VMEM((128,), f32) for _ in range(3)]
            + [pltpu.VMEM((128,), i32) for _ in range(3)]
            + [pltpu.VMEM((128,), f32)]
            + [pltpu.VMEM((128,), i32) for _ in range(3)]
            + [pltpu.VMEM((128,), f32) for _ in range(9)]
            + [pltpu.VMEM((128,), f32) for _ in range(3)]
            + [pltpu.SemaphoreType.DMA]
        ),
    )(*args_o, *args_s)


def _dist_body(ot_ref, st_ref, prob_ref, out_ref, colmin_ref, osum_ref):
    i = pl.program_id(0)
    ob = ot_ref[...]
    st = st_ref[...]
    on = jnp.sum(ob * ob, axis=0)
    sn = jnp.sum(st * st, axis=0)
    cross = jax.lax.dot_general(
        ob, st, (((0,), (0,)), ((), ())),
        preferred_element_type=jnp.float32,
        precision=jax.lax.Precision.HIGHEST)
    d2 = jnp.maximum(on[:, None] + sn[None, :] - 2.0 * cross, 0.0)

    @pl.when(i == 0)
    def _init():
        colmin_ref[...] = jnp.full((_S,), jnp.inf, jnp.float32)
        osum_ref[0] = 0.0

    colmin_ref[...] = jnp.minimum(colmin_ref[...], jnp.min(d2, axis=0))
    osum_ref[0] += jnp.sum(jnp.sqrt(jnp.min(d2, axis=1)))

    @pl.when(i == pl.num_programs(0) - 1)
    def _fin():
        mean_o2s = osum_ref[0] / _S
        mean_s2o = jnp.sum(jnp.sqrt(colmin_ref[...])) / _S
        mean_p = jnp.sum(prob_ref[...]) / _NP
        out_ref[0, 0] = mean_s2o * mean_p + mean_o2s


def _min_dist_loss(ot, st, prob_pad, interpret=False):
    return pl.pallas_call(
        _dist_body,
        grid=(_S // _BLK,),
        in_specs=[
            pl.BlockSpec((3, _BLK), lambda i: (0, i)),
            pl.BlockSpec((3, _S), lambda i: (0, 0)),
            pl.BlockSpec((80, 128), lambda i: (0, 0)),
        ],
        out_specs=pl.BlockSpec(memory_space=pltpu.SMEM),
        out_shape=jax.ShapeDtypeStruct((1, 1), jnp.float32),
        scratch_shapes=[
            pltpu.VMEM((_S,), jnp.float32),
            pltpu.SMEM((1,), jnp.float32),
        ],
        interpret=interpret,
    )(ot, st, prob_pad)


def kernel(original_vertices, original_faces, simplified_vertices,
           simplified_faces, sample_prob):
    ko1, ko2, ko3 = jax.random.split(jax.random.key(1), 3)
    ks1, ks2, ks3 = jax.random.split(jax.random.key(2), 3)
    ro = 1.0 - jax.random.uniform(ko1, (_S,), dtype=jnp.float32)
    rs = 1.0 - jax.random.uniform(ks1, (_S,), dtype=jnp.float32)
    uo = jax.random.uniform(ko2, (_S,), dtype=jnp.float32)
    vo = jax.random.uniform(ko3, (_S,), dtype=jnp.float32)
    us = jax.random.uniform(ks2, (_S,), dtype=jnp.float32)
    vs = jax.random.uniform(ks3, (_S,), dtype=jnp.float32)

    vo_comp = tuple(original_vertices[:, c] for c in range(3))
    vs_comp = tuple(simplified_vertices[:, c] for c in range(3))
    fo_perm = jnp.pad(original_faces, ((0, _NW * _PWO - _FO), (0, 0))) \
        .reshape(-1, _GROUP, 3).transpose(0, 2, 1).reshape(-1)
    fs_perm = jnp.pad(simplified_faces, ((0, _NW * _PWS - _FS), (0, 0))) \
        .reshape(-1, _GROUP, 3).transpose(0, 2, 1).reshape(-1)

    cumo, cums = _sc_areas(fo_perm, vo_comp, fs_perm, vs_comp)

    glob_o, r_o = _assemble_glob(cumo, _PWO, ro)
    glob_s, r_s = _assemble_glob(cums, _PWS, rs)

    fo_tabs = tuple(original_faces[:, k] for k in range(3))
    fs_tabs = tuple(simplified_faces[:, k] for k in range(3))
    pts_o, pts_s = _sc_sample(
        (glob_o, *fo_tabs, *vo_comp, r_o, uo, vo),
        (glob_s, *fs_tabs, *vs_comp, r_s, us, vs))

    prob_pad = jnp.concatenate(
        [sample_prob, jnp.zeros((80 * 128 - _NP,), jnp.float32)]).reshape(80, 128)
    loss = _min_dist_loss(pts_o.reshape(3, _S), pts_s.reshape(3, _S), prob_pad)
    return loss[0, 0]


# speedup vs baseline: 14.3931x; 1.3482x over previous
import functools

import jax
import jax.numpy as jnp
from jax import lax
from jax.experimental import pallas as pl
from jax.experimental.pallas import tpu as pltpu
from jax.experimental.pallas import tpu_sc as plsc

_S = 4096
_BLK = 512
_NP = 10000
_NW = 32

_FO, _PWO = 200000, 6272
_FS, _PWS = 20000, 640
_GROUP = 32


def _compute_group_areas(g, bufs, cum_ref):
    for sub in range(2):
        v = [[bufs[c][pl.ds(k * 32 + sub * 16, 16)] for c in range(3)]
             for k in range(3)]
        e1 = [v[1][c] - v[0][c] for c in range(3)]
        e2 = [v[2][c] - v[0][c] for c in range(3)]
        nx = e1[1] * e2[2] - e1[2] * e2[1]
        ny = e1[2] * e2[0] - e1[0] * e2[2]
        nz = e1[0] * e2[1] - e1[1] * e2[0]
        q = nx * nx + ny * ny + nz * nz
        y = plsc.bitcast(jnp.int32(0x5F3759DF) - (plsc.bitcast(q, jnp.int32) >> 1),
                         jnp.float32)
        for _ in range(3):
            y = y * (1.5 - 0.5 * q * y * y)
        area = jnp.where(q > 0.0, 0.5 * q * y, 0.0)
        cum_ref[pl.ds(g * _GROUP + sub * 16, 16)] = area


def _area_section(faces_hbm, vcomp_hbm, cum_hbm, idx_v, bufs, sems, cum_v,
                  wid, n_per_wid):
    n_idx = 3 * n_per_wid
    n_groups = n_per_wid // _GROUP
    pltpu.sync_copy(faces_hbm.at[pl.ds(wid * n_idx, n_idx)],
                    idx_v.at[pl.ds(0, n_idx)])

    nbuf = len(bufs)

    def issue(g, b):
        for c in range(3):
            pltpu.async_copy(vcomp_hbm[c].at[idx_v.at[pl.ds(g * 96, 96)]],
                             bufs[b][c], sems[b])

    def wait(b):
        for c in range(3):
            pltpu.make_async_copy(vcomp_hbm[c].at[pl.ds(0, 96)], bufs[b][c],
                                  sems[b]).wait()

    for b in range(nbuf):
        issue(b, b)

    def outer(q, _):
        for b in range(nbuf):
            g = q * nbuf + b
            wait(b)
            _compute_group_areas(g, bufs[b], cum_v)

            @pl.when(g + nbuf < n_groups)
            def _():
                issue(g + nbuf, b)
        return 0

    lax.fori_loop(0, n_groups // nbuf, outer, 0)

    def cs_body(i, carry):
        vv = cum_v[pl.ds(i * 16, 16)]
        c2 = jnp.cumsum(vv) + carry
        cum_v[pl.ds(i * 16, 16)] = c2
        return jnp.max(c2)

    lax.fori_loop(0, n_per_wid // 16, cs_body, jnp.float32(0.0))
    pltpu.sync_copy(cum_v.at[pl.ds(0, n_per_wid)],
                    cum_hbm.at[pl.ds(wid * n_per_wid, n_per_wid)])


def _sc_areas_body(fo_hbm, vxo, vyo, vzo, fs_hbm, vxs, vys, vzs,
                   cumo_hbm, cums_hbm,
                   idx_v, b0x, b0y, b0z, b1x, b1y, b1z, b2x, b2y, b2z,
                   b3x, b3y, b3z, cum_v, sem0, sem1, sem2, sem3):
    wid = lax.axis_index("s") * 2 + lax.axis_index("c")
    bufs = ((b0x, b0y, b0z), (b1x, b1y, b1z), (b2x, b2y, b2z),
            (b3x, b3y, b3z))
    sems = (sem0, sem1, sem2, sem3)
    _area_section(fo_hbm, (vxo, vyo, vzo), cumo_hbm, idx_v, bufs, sems, cum_v,
                  wid, _PWO)
    _area_section(fs_hbm, (vxs, vys, vzs), cums_hbm, idx_v, bufs, sems, cum_v,
                  wid, _PWS)


def _sc_areas(fo_perm, vo_comp, fs_perm, vs_comp):
    mesh = plsc.VectorSubcoreMesh(core_axis_name="c", subcore_axis_name="s")
    return pl.kernel(
        _sc_areas_body,
        out_type=(jax.ShapeDtypeStruct((_NW * _PWO,), jnp.float32),
                  jax.ShapeDtypeStruct((_NW * _PWS,), jnp.float32)),
        mesh=mesh,
        compiler_params=pltpu.CompilerParams(needs_layout_passes=False),
        scratch_types=(
            [pltpu.VMEM((3 * _PWO,), jnp.int32)]
            + [pltpu.VMEM((96,), jnp.float32) for _ in range(12)]
            + [pltpu.VMEM((_PWO,), jnp.float32)]
            + [pltpu.SemaphoreType.DMA for _ in range(4)]
        ),
    )(fo_perm, *vo_comp, fs_perm, *vs_comp)


def _assemble_glob(cum, n_per_wid, r_unit):
    co = cum.reshape(_NW, n_per_wid)
    tot = co[:, -1]
    off = jnp.concatenate([jnp.zeros((1,), jnp.float32), jnp.cumsum(tot)[:-1]])
    glob = (co + off[:, None]).reshape(-1)
    total = off[-1] + tot[-1]
    return glob, total * r_unit


def _sample_section(glob2d_hbm, coarse_hbm, ftabs, vtabs, r_hbm, u_hbm, v_hbm,
                    pts_hbm, scr, sem, wid, nc_steps, n_win):
    (r_v, u_v, v_v, lo_v, hi_v, mid_v, coarse_v, win_v, id0, id1, id2,
     cbufs, px_v, py_v, pz_v) = scr
    base = wid * 128
    pltpu.sync_copy(r_hbm.at[pl.ds(base, 128)], r_v)
    pltpu.sync_copy(u_hbm.at[pl.ds(base, 128)], u_v)
    pltpu.sync_copy(v_hbm.at[pl.ds(base, 128)], v_v)
    pltpu.sync_copy(coarse_hbm.at[pl.ds(0, n_win)], coarse_v.at[pl.ds(0, n_win)])
    for j in range(8):
        s = pl.ds(j * 16, 16)
        lo_v[s] = jnp.zeros((16,), jnp.int32)
        hi_v[s] = jnp.full((16,), n_win, jnp.int32)

    def cstep(t, _):
        for j in range(8):
            s = pl.ds(j * 16, 16)
            mid = (lo_v[s] + hi_v[s]) // 2
            val = plsc.load_gather(coarse_v, [mid])
            pred = val < r_v[s]
            lo_v[s] = jnp.where(pred, mid + 1, lo_v[s])
            hi_v[s] = jnp.where(pred, hi_v[s], mid)
        return 0

    lax.fori_loop(0, nc_steps, cstep, 0)

    pltpu.async_copy(glob2d_hbm.at[lo_v], win_v, sem).wait()

    for j in range(8):
        s = pl.ds(j * 16, 16)
        rows = lax.iota(jnp.int32, 16) + j * 16
        lo = jnp.zeros((16,), jnp.int32)
        hi = jnp.full((16,), 128, jnp.int32)
        for _ in range(7):
            mid = (lo + hi) // 2
            val = plsc.load_gather(win_v, [rows, mid])
            pred = val < r_v[s]
            lo = jnp.where(pred, mid + 1, lo)
            hi = jnp.where(pred, hi, mid)
        lo_v[s] = lo_v[s] * 128 + lo

    ids = (id0, id1, id2)
    for k in range(3):
        pltpu.async_copy(ftabs[k].at[lo_v], ids[k], sem)
    for k in range(3):
        pltpu.make_async_copy(ftabs[k].at[pl.ds(0, 128)], ids[k], sem).wait()
    for k in range(3):
        for c in range(3):
            pltpu.async_copy(vtabs[c].at[ids[k]], cbufs[k][c], sem)
    for k in range(3):
        for c in range(3):
            pltpu.make_async_copy(vtabs[c].at[pl.ds(0, 128)], cbufs[k][c],
                                  sem).wait()

    outs = (px_v, py_v, pz_v)
    for j in range(8):
        s = pl.ds(j * 16, 16)
        u = u_v[s]
        v = v_v[s]
        w = 1.0 - u - v
        mask = w < 0
        u = jnp.where(mask, u + w, u)
        v = jnp.where(mask, 1.0 - u, v)
        w = jnp.where(mask, 0.0, w)
        for c in range(3):
            outs[c][s] = (u * cbufs[0][c][s] + v * cbufs[1][c][s]
                          + w * cbufs[2][c][s])
    for c in range(3):
        pltpu.sync_copy(outs[c], pts_hbm.at[pl.ds(c * _S + base, 128)])


def _sc_sample_body(glob2do, coarseo, f0o, f1o, f2o, vxo, vyo, vzo, ro, uo, vo,
                    glob2ds, coarses, f0s, f1s, f2s, vxs, vys, vzs, rs, us, vs,
                    ptso_hbm, ptss_hbm,
                    r_v, u_v, v_v, lo_v, hi_v, mid_v, coarse_v, win_v,
                    id0, id1, id2,
                    c00, c01, c02, c10, c11, c12, c20, c21, c22,
                    px_v, py_v, pz_v, sem):
    wid = lax.axis_index("s") * 2 + lax.axis_index("c")
    cbufs = ((c00, c01, c02), (c10, c11, c12), (c20, c21, c22))
    scr = (r_v, u_v, v_v, lo_v, hi_v, mid_v, coarse_v, win_v, id0, id1, id2,
           cbufs, px_v, py_v, pz_v)
    _sample_section(glob2do, coarseo, (f0o, f1o, f2o), (vxo, vyo, vzo),
                    ro, uo, vo, ptso_hbm, scr, sem, wid, 11, _NW * _PWO // 128)
    _sample_section(glob2ds, coarses, (f0s, f1s, f2s), (vxs, vys, vzs),
                    rs, us, vs, ptss_hbm, scr, sem, wid, 8, _NW * _PWS // 128)


def _sc_sample(args_o, args_s):
    mesh = plsc.VectorSubcoreMesh(core_axis_name="c", subcore_axis_name="s")
    f32 = jnp.float32
    i32 = jnp.int32
    return pl.kernel(
        _sc_sample_body,
        out_type=(jax.ShapeDtypeStruct((3 * _S,), f32),
                  jax.ShapeDtypeStruct((3 * _S,), f32)),
        mesh=mesh,
        compiler_params=pltpu.CompilerParams(needs_layout_passes=False),
        scratch_types=(
            [pltpu.VMEM((128,), f32) for _ in range(3)]
            + [pltpu.VMEM((128,), i32) for _ in range(3)]
            + [pltpu.VMEM((_NW * _PWO // 128,), f32)]
            + [pltpu.VMEM((128, 128), f32)]
            + [pltpu.VMEM((128,), i32) for _ in range(3)]
            + [pltpu.VMEM((128,), f32) for _ in range(9)]
            + [pltpu.VMEM((128,), f32) for _ in range(3)]
            + [pltpu.SemaphoreType.DMA]
        ),
    )(*args_o, *args_s)


def _dist_body(ot_ref, st_ref, prob_ref, out_ref, colmin_ref, osum_ref):
    i = pl.program_id(0)
    ob = ot_ref[...]
    st = st_ref[...]
    on = jnp.sum(ob * ob, axis=0)
    sn = jnp.sum(st * st, axis=0)
    cross = jax.lax.dot_general(
        ob, st, (((0,), (0,)), ((), ())),
        preferred_element_type=jnp.float32,
        precision=jax.lax.Precision.HIGHEST)
    d2 = jnp.maximum(on[:, None] + sn[None, :] - 2.0 * cross, 0.0)

    @pl.when(i == 0)
    def _init():
        colmin_ref[...] = jnp.full((_S,), jnp.inf, jnp.float32)
        osum_ref[0] = 0.0

    colmin_ref[...] = jnp.minimum(colmin_ref[...], jnp.min(d2, axis=0))
    osum_ref[0] += jnp.sum(jnp.sqrt(jnp.min(d2, axis=1)))

    @pl.when(i == pl.num_programs(0) - 1)
    def _fin():
        mean_o2s = osum_ref[0] / _S
        mean_s2o = jnp.sum(jnp.sqrt(colmin_ref[...])) / _S
        mean_p = jnp.sum(prob_ref[...]) / _NP
        out_ref[0, 0] = mean_s2o * mean_p + mean_o2s


def _min_dist_loss(ot, st, prob_pad, interpret=False):
    return pl.pallas_call(
        _dist_body,
        grid=(_S // _BLK,),
        in_specs=[
            pl.BlockSpec((3, _BLK), lambda i: (0, i)),
            pl.BlockSpec((3, _S), lambda i: (0, 0)),
            pl.BlockSpec((80, 128), lambda i: (0, 0)),
        ],
        out_specs=pl.BlockSpec(memory_space=pltpu.SMEM),
        out_shape=jax.ShapeDtypeStruct((1, 1), jnp.float32),
        scratch_shapes=[
            pltpu.VMEM((_S,), jnp.float32),
            pltpu.SMEM((1,), jnp.float32),
        ],
        interpret=interpret,
    )(ot, st, prob_pad)


def kernel(original_vertices, original_faces, simplified_vertices,
           simplified_faces, sample_prob):
    ko1, ko2, ko3 = jax.random.split(jax.random.key(1), 3)
    ks1, ks2, ks3 = jax.random.split(jax.random.key(2), 3)
    ro = 1.0 - jax.random.uniform(ko1, (_S,), dtype=jnp.float32)
    rs = 1.0 - jax.random.uniform(ks1, (_S,), dtype=jnp.float32)
    uo = jax.random.uniform(ko2, (_S,), dtype=jnp.float32)
    vo = jax.random.uniform(ko3, (_S,), dtype=jnp.float32)
    us = jax.random.uniform(ks2, (_S,), dtype=jnp.float32)
    vs = jax.random.uniform(ks3, (_S,), dtype=jnp.float32)

    vo_comp = tuple(original_vertices[:, c] for c in range(3))
    vs_comp = tuple(simplified_vertices[:, c] for c in range(3))
    fo_perm = jnp.pad(original_faces, ((0, _NW * _PWO - _FO), (0, 0))) \
        .reshape(-1, _GROUP, 3).transpose(0, 2, 1).reshape(-1)
    fs_perm = jnp.pad(simplified_faces, ((0, _NW * _PWS - _FS), (0, 0))) \
        .reshape(-1, _GROUP, 3).transpose(0, 2, 1).reshape(-1)

    cumo, cums = _sc_areas(fo_perm, vo_comp, fs_perm, vs_comp)

    glob_o, r_o = _assemble_glob(cumo, _PWO, ro)
    glob_s, r_s = _assemble_glob(cums, _PWS, rs)

    fo_tabs = tuple(original_faces[:, k] for k in range(3))
    fs_tabs = tuple(simplified_faces[:, k] for k in range(3))
    glob2d_o = glob_o.reshape(-1, 128)
    glob2d_s = glob_s.reshape(-1, 128)
    pts_o, pts_s = _sc_sample(
        (glob2d_o, glob2d_o[:, -1], *fo_tabs, *vo_comp, r_o, uo, vo),
        (glob2d_s, glob2d_s[:, -1], *fs_tabs, *vs_comp, r_s, us, vs))

    prob_pad = jnp.concatenate(
        [sample_prob, jnp.zeros((80 * 128 - _NP,), jnp.float32)]).reshape(80, 128)
    loss = _min_dist_loss(pts_o.reshape(3, _S), pts_s.reshape(3, _S), prob_pad)
    return loss[0, 0]


# speedup vs baseline: 15.8555x; 1.1016x over previous
import functools

import jax
import jax.numpy as jnp
from jax import lax
from jax.experimental import pallas as pl
from jax.experimental.pallas import tpu as pltpu
from jax.experimental.pallas import tpu_sc as plsc

_S = 4096
_BLK = 512
_NP = 10000
_NW = 32

_FO, _PWO = 200000, 6272
_FS, _PWS = 20000, 640
_GROUP = 32


def _compute_group_areas(g, bufs, cum_ref):
    for sub in range(8):
        v = [[bufs[k][c][pl.ds(sub * 16, 16)] for c in range(3)]
             for k in range(3)]
        e1 = [v[1][c] - v[0][c] for c in range(3)]
        e2 = [v[2][c] - v[0][c] for c in range(3)]
        nx = e1[1] * e2[2] - e1[2] * e2[1]
        ny = e1[2] * e2[0] - e1[0] * e2[2]
        nz = e1[0] * e2[1] - e1[1] * e2[0]
        q = nx * nx + ny * ny + nz * nz
        y = plsc.bitcast(jnp.int32(0x5F3759DF) - (plsc.bitcast(q, jnp.int32) >> 1),
                         jnp.float32)
        for _ in range(3):
            y = y * (1.5 - 0.5 * q * y * y)
        area = jnp.where(q > 0.0, 0.5 * q * y, 0.0)
        cum_ref[pl.ds(g * 128 + sub * 16, 16)] = area


def _area_section(ftab_hbm, vcomp_hbm, cum_hbm, idx_vs, bufs, sems, cum_v,
                  wid, n_per_wid, nbuf):
    n_groups = n_per_wid // 128
    for k in range(3):
        pltpu.sync_copy(ftab_hbm[k].at[pl.ds(wid * n_per_wid, n_per_wid)],
                        idx_vs[k].at[pl.ds(0, n_per_wid)])

    def issue(g, b):
        for k in range(3):
            idx = idx_vs[k].at[pl.ds(g * 128, 128)]
            for c in range(3):
                pltpu.async_copy(vcomp_hbm[c].at[idx], bufs[b][k][c], sems[b])

    def wait(b):
        for k in range(3):
            for c in range(3):
                pltpu.make_async_copy(vcomp_hbm[c].at[pl.ds(0, 128)],
                                      bufs[b][k][c], sems[b]).wait()

    for b in range(nbuf):
        issue(b, b)

    def outer(q, _):
        for b in range(nbuf):
            g = q * nbuf + b
            wait(b)
            _compute_group_areas(g, bufs[b], cum_v)

            @pl.when(g + nbuf < n_groups)
            def _():
                issue(g + nbuf, b)
        return 0

    lax.fori_loop(0, n_groups // nbuf, outer, 0)

    def cs_body(i, carry):
        vv = cum_v[pl.ds(i * 16, 16)]
        c2 = jnp.cumsum(vv) + carry
        cum_v[pl.ds(i * 16, 16)] = c2
        return jnp.max(c2)

    lax.fori_loop(0, n_per_wid // 16, cs_body, jnp.float32(0.0))
    pltpu.sync_copy(cum_v.at[pl.ds(0, n_per_wid)],
                    cum_hbm.at[pl.ds(wid * n_per_wid, n_per_wid)])


_NBUF = 7


def _sc_areas_body(f0o, f1o, f2o, vxo, vyo, vzo, f0s, f1s, f2s, vxs, vys, vzs,
                   cumo_hbm, cums_hbm, *scr):
    wid = lax.axis_index("s") * 2 + lax.axis_index("c")
    idx_vs = scr[0:3]
    bufs = tuple(
        tuple(tuple(scr[3 + b * 9 + k * 3 + c] for c in range(3))
              for k in range(3)) for b in range(_NBUF))
    cum_v = scr[3 + 9 * _NBUF]
    sems = scr[4 + 9 * _NBUF:]
    _area_section((f0o, f1o, f2o), (vxo, vyo, vzo), cumo_hbm, idx_vs,
                  bufs, sems, cum_v, wid, _PWO, 7)
    _area_section((f0s, f1s, f2s), (vxs, vys, vzs), cums_hbm, idx_vs,
                  bufs, sems, cum_v, wid, _PWS, 5)


def _sc_areas(fo_tabs, vo_comp, fs_tabs, vs_comp):
    mesh = plsc.VectorSubcoreMesh(core_axis_name="c", subcore_axis_name="s")
    return pl.kernel(
        _sc_areas_body,
        out_type=(jax.ShapeDtypeStruct((_NW * _PWO,), jnp.float32),
                  jax.ShapeDtypeStruct((_NW * _PWS,), jnp.float32)),
        mesh=mesh,
        compiler_params=pltpu.CompilerParams(needs_layout_passes=False),
        scratch_types=(
            [pltpu.VMEM((_PWO,), jnp.int32) for _ in range(3)]
            + [pltpu.VMEM((128,), jnp.float32) for _ in range(9 * _NBUF)]
            + [pltpu.VMEM((_PWO,), jnp.float32)]
            + [pltpu.SemaphoreType.DMA for _ in range(_NBUF)]
        ),
    )(*fo_tabs, *vo_comp, *fs_tabs, *vs_comp)


def _assemble_glob(cum, n_per_wid, r_unit):
    co = cum.reshape(_NW, n_per_wid)
    tot = co[:, -1]
    off = jnp.concatenate([jnp.zeros((1,), jnp.float32), jnp.cumsum(tot)[:-1]])
    glob = (co + off[:, None]).reshape(-1)
    total = off[-1] + tot[-1]
    return glob, total * r_unit


def _sample_section(glob2d_hbm, coarse_hbm, ftabs, vtabs, r_hbm, u_hbm, v_hbm,
                    pts_hbm, scr, sem, wid, nc_steps, n_win):
    (r_v, u_v, v_v, lo_v, hi_v, mid_v, coarse_v, win_v, id0, id1, id2,
     cbufs, px_v, py_v, pz_v) = scr
    base = wid * 128
    pltpu.sync_copy(r_hbm.at[pl.ds(base, 128)], r_v)
    pltpu.sync_copy(u_hbm.at[pl.ds(base, 128)], u_v)
    pltpu.sync_copy(v_hbm.at[pl.ds(base, 128)], v_v)
    pltpu.sync_copy(coarse_hbm.at[pl.ds(0, n_win)], coarse_v.at[pl.ds(0, n_win)])
    for j in range(8):
        s = pl.ds(j * 16, 16)
        lo_v[s] = jnp.zeros((16,), jnp.int32)
        hi_v[s] = jnp.full((16,), n_win, jnp.int32)

    def cstep(t, _):
        for j in range(8):
            s = pl.ds(j * 16, 16)
            mid = (lo_v[s] + hi_v[s]) // 2
            val = plsc.load_gather(coarse_v, [mid])
            pred = val < r_v[s]
            lo_v[s] = jnp.where(pred, mid + 1, lo_v[s])
            hi_v[s] = jnp.where(pred, hi_v[s], mid)
        return 0

    lax.fori_loop(0, nc_steps, cstep, 0)

    pltpu.async_copy(glob2d_hbm.at[lo_v], win_v, sem).wait()

    for j in range(8):
        s = pl.ds(j * 16, 16)
        rows = lax.iota(jnp.int32, 16) + j * 16
        lo = jnp.zeros((16,), jnp.int32)
        hi = jnp.full((16,), 128, jnp.int32)
        for _ in range(7):
            mid = (lo + hi) // 2
            val = plsc.load_gather(win_v, [rows, mid])
            pred = val < r_v[s]
            lo = jnp.where(pred, mid + 1, lo)
            hi = jnp.where(pred, hi, mid)
        lo_v[s] = lo_v[s] * 128 + lo

    ids = (id0, id1, id2)
    for k in range(3):
        pltpu.async_copy(ftabs[k].at[lo_v], ids[k], sem)
    for k in range(3):
        pltpu.make_async_copy(ftabs[k].at[pl.ds(0, 128)], ids[k], sem).wait()
    for k in range(3):
        for c in range(3):
            pltpu.async_copy(vtabs[c].at[ids[k]], cbufs[k][c], sem)
    for k in range(3):
        for c in range(3):
            pltpu.make_async_copy(vtabs[c].at[pl.ds(0, 128)], cbufs[k][c],
                                  sem).wait()

    outs = (px_v, py_v, pz_v)
    for j in range(8):
        s = pl.ds(j * 16, 16)
        u = u_v[s]
        v = v_v[s]
        w = 1.0 - u - v
        mask = w < 0
        u = jnp.where(mask, u + w, u)
        v = jnp.where(mask, 1.0 - u, v)
        w = jnp.where(mask, 0.0, w)
        for c in range(3):
            outs[c][s] = (u * cbufs[0][c][s] + v * cbufs[1][c][s]
                          + w * cbufs[2][c][s])
    for c in range(3):
        pltpu.sync_copy(outs[c], pts_hbm.at[pl.ds(c * _S + base, 128)])


def _sc_sample_body(glob2do, coarseo, f0o, f1o, f2o, vxo, vyo, vzo, ro, uo, vo,
                    glob2ds, coarses, f0s, f1s, f2s, vxs, vys, vzs, rs, us, vs,
                    ptso_hbm, ptss_hbm,
                    r_v, u_v, v_v, lo_v, hi_v, mid_v, coarse_v, win_v,
                    id0, id1, id2,
                    c00, c01, c02, c10, c11, c12, c20, c21, c22,
                    px_v, py_v, pz_v, sem):
    wid = lax.axis_index("s") * 2 + lax.axis_index("c")
    cbufs = ((c00, c01, c02), (c10, c11, c12), (c20, c21, c22))
    scr = (r_v, u_v, v_v, lo_v, hi_v, mid_v, coarse_v, win_v, id0, id1, id2,
           cbufs, px_v, py_v, pz_v)
    _sample_section(glob2do, coarseo, (f0o, f1o, f2o), (vxo, vyo, vzo),
                    ro, uo, vo, ptso_hbm, scr, sem, wid, 11, _NW * _PWO // 128)
    _sample_section(glob2ds, coarses, (f0s, f1s, f2s), (vxs, vys, vzs),
                    rs, us, vs, ptss_hbm, scr, sem, wid, 8, _NW * _PWS // 128)


def _sc_sample(args_o, args_s):
    mesh = plsc.VectorSubcoreMesh(core_axis_name="c", subcore_axis_name="s")
    f32 = jnp.float32
    i32 = jnp.int32
    return pl.kernel(
        _sc_sample_body,
        out_type=(jax.ShapeDtypeStruct((3 * _S,), f32),
                  jax.ShapeDtypeStruct((3 * _S,), f32)),
        mesh=mesh,
        compiler_params=pltpu.CompilerParams(needs_layout_passes=False),
        scratch_types=(
            [pltpu.VMEM((128,), f32) for _ in range(3)]
            + [pltpu.VMEM((128,), i32) for _ in range(3)]
            + [pltpu.VMEM((_NW * _PWO // 128,), f32)]
            + [pltpu.VMEM((128, 128), f32)]
            + [pltpu.VMEM((128,), i32) for _ in range(3)]
            + [pltpu.VMEM((128,), f32) for _ in range(9)]
            + [pltpu.VMEM((128,), f32) for _ in range(3)]
            + [pltpu.SemaphoreType.DMA]
        ),
    )(*args_o, *args_s)


def _dist_body(ot_ref, st_ref, prob_ref, out_ref, colmin_ref, osum_ref):
    i = pl.program_id(0)
    ob = ot_ref[...]
    st = st_ref[...]
    on = jnp.sum(ob * ob, axis=0)
    sn = jnp.sum(st * st, axis=0)
    cross = jax.lax.dot_general(
        ob, st, (((0,), (0,)), ((), ())),
        preferred_element_type=jnp.float32,
        precision=jax.lax.Precision.HIGHEST)
    d2 = jnp.maximum(on[:, None] + sn[None, :] - 2.0 * cross, 0.0)

    @pl.when(i == 0)
    def _init():
        colmin_ref[...] = jnp.full((_S,), jnp.inf, jnp.float32)
        osum_ref[0] = 0.0

    colmin_ref[...] = jnp.minimum(colmin_ref[...], jnp.min(d2, axis=0))
    osum_ref[0] += jnp.sum(jnp.sqrt(jnp.min(d2, axis=1)))

    @pl.when(i == pl.num_programs(0) - 1)
    def _fin():
        mean_o2s = osum_ref[0] / _S
        mean_s2o = jnp.sum(jnp.sqrt(colmin_ref[...])) / _S
        mean_p = jnp.sum(prob_ref[...]) / _NP
        out_ref[0, 0] = mean_s2o * mean_p + mean_o2s


def _min_dist_loss(ot, st, prob_pad, interpret=False):
    return pl.pallas_call(
        _dist_body,
        grid=(_S // _BLK,),
        in_specs=[
            pl.BlockSpec((3, _BLK), lambda i: (0, i)),
            pl.BlockSpec((3, _S), lambda i: (0, 0)),
            pl.BlockSpec((80, 128), lambda i: (0, 0)),
        ],
        out_specs=pl.BlockSpec(memory_space=pltpu.SMEM),
        out_shape=jax.ShapeDtypeStruct((1, 1), jnp.float32),
        scratch_shapes=[
            pltpu.VMEM((_S,), jnp.float32),
            pltpu.SMEM((1,), jnp.float32),
        ],
        interpret=interpret,
    )(ot, st, prob_pad)


def kernel(original_vertices, original_faces, simplified_vertices,
           simplified_faces, sample_prob):
    ko1, ko2, ko3 = jax.random.split(jax.random.key(1), 3)
    ks1, ks2, ks3 = jax.random.split(jax.random.key(2), 3)
    ro = 1.0 - jax.random.uniform(ko1, (_S,), dtype=jnp.float32)
    rs = 1.0 - jax.random.uniform(ks1, (_S,), dtype=jnp.float32)
    uo = jax.random.uniform(ko2, (_S,), dtype=jnp.float32)
    vo = jax.random.uniform(ko3, (_S,), dtype=jnp.float32)
    us = jax.random.uniform(ks2, (_S,), dtype=jnp.float32)
    vs = jax.random.uniform(ks3, (_S,), dtype=jnp.float32)

    vo_comp = tuple(original_vertices[:, c] for c in range(3))
    vs_comp = tuple(simplified_vertices[:, c] for c in range(3))
    fo_tabs = tuple(original_faces[:, k] for k in range(3))
    fs_tabs = tuple(simplified_faces[:, k] for k in range(3))
    fo_tabs_pad = tuple(jnp.pad(t, (0, _NW * _PWO - _FO)) for t in fo_tabs)
    fs_tabs_pad = tuple(jnp.pad(t, (0, _NW * _PWS - _FS)) for t in fs_tabs)

    cumo, cums = _sc_areas(fo_tabs_pad, vo_comp, fs_tabs_pad, vs_comp)

    glob_o, r_o = _assemble_glob(cumo, _PWO, ro)
    glob_s, r_s = _assemble_glob(cums, _PWS, rs)
    glob2d_o = glob_o.reshape(-1, 128)
    glob2d_s = glob_s.reshape(-1, 128)
    pts_o, pts_s = _sc_sample(
        (glob2d_o, glob2d_o[:, -1], *fo_tabs, *vo_comp, r_o, uo, vo),
        (glob2d_s, glob2d_s[:, -1], *fs_tabs, *vs_comp, r_s, us, vs))

    prob_pad = jnp.concatenate(
        [sample_prob, jnp.zeros((80 * 128 - _NP,), jnp.float32)]).reshape(80, 128)
    loss = _min_dist_loss(pts_o.reshape(3, _S), pts_s.reshape(3, _S), prob_pad)
    return loss[0, 0]
